# baseline (speedup 1.0000x reference)
import functools

import jax
import jax.numpy as jnp
from jax import lax
from jax.experimental import pallas as pl
from jax.experimental.pallas import tpu as pltpu
from jax.experimental.pallas import tpu_sc as plsc

_CHUNK = 128


@functools.cache
def _build(B, V, D):
    info = plsc.get_sparse_core_info()
    NC, NS = info.num_cores, info.num_subcores
    NW = NC * NS
    b_per_w = B // NW
    n_chunks = b_per_w // _CHUNK
    mesh = plsc.VectorSubcoreMesh(core_axis_name="c", subcore_axis_name="s")

    @functools.partial(
        pl.kernel,
        mesh=mesh,
        out_type=jax.ShapeDtypeStruct((B, D), jnp.float32),
        scratch_types=[
            pltpu.VMEM((n_chunks, _CHUNK), jnp.int32),
            pltpu.VMEM((n_chunks, _CHUNK, D), jnp.float32),
            pltpu.SemaphoreType.DMA((n_chunks,)),
            pltpu.SemaphoreType.DMA,
        ],
    )
    def k(idx_hbm, table_hbm, out_hbm, idx_v, rows_v, gsem, wsem):
        wid = lax.axis_index("s") * NC + lax.axis_index("c")
        base = wid * b_per_w
        pltpu.sync_copy(idx_hbm.at[wid], idx_v)
        gathers = [
            pltpu.async_copy(
                table_hbm.at[idx_v.at[j]],
                rows_v.at[j],
                gsem.at[j],
            )
            for j in range(n_chunks)
        ]
        writes = []
        for j in range(n_chunks):
            gathers[j].wait()
            writes.append(
                pltpu.async_copy(
                    rows_v.at[j],
                    out_hbm.at[pl.ds(base + j * _CHUNK, _CHUNK)],
                    wsem,
                )
            )
        for w in writes:
            w.wait()

    def run(x, table):
        idx = x.astype(jnp.int32).reshape(NW, n_chunks, _CHUNK)
        out = k(idx, table)
        return out.reshape(B, 1, D)

    return run


def kernel(x, table):
    B = x.shape[0]
    V, D = table.shape
    return _build(B, V, D)(x, table)


# speedup vs baseline: 1.0315x; 1.0315x over previous
import functools

import jax
import jax.numpy as jnp
from jax import lax
from jax.experimental import pallas as pl
from jax.experimental.pallas import tpu as pltpu
from jax.experimental.pallas import tpu_sc as plsc


@functools.cache
def _build(B, V, D):
    info = plsc.get_sparse_core_info()
    NC, NS = info.num_cores, info.num_subcores
    NW = NC * NS
    b_per_w = B // NW
    mesh = plsc.VectorSubcoreMesh(core_axis_name="c", subcore_axis_name="s")

    @functools.partial(
        pl.kernel,
        mesh=mesh,
        out_type=jax.ShapeDtypeStruct((B, D), jnp.float32),
        scratch_types=[
            pltpu.VMEM((1, b_per_w), jnp.int32),
            pltpu.VMEM((b_per_w, D), jnp.float32),
            pltpu.SemaphoreType.DMA,
        ],
    )
    def k(idx_hbm, table_hbm, out_hbm, idx_v, rows_v, sem):
        wid = lax.axis_index("s") * NC + lax.axis_index("c")
        base = wid * b_per_w
        pltpu.sync_copy(idx_hbm.at[wid], idx_v)
        pltpu.async_copy(table_hbm.at[idx_v.at[0]], rows_v, sem).wait()
        pltpu.sync_copy(rows_v, out_hbm.at[pl.ds(base, b_per_w)])

    def run(x, table):
        idx = x.astype(jnp.int32).reshape(NW, 1, b_per_w)
        out = k(idx, table)
        return out.reshape(B, 1, D)

    return run


def kernel(x, table):
    B = x.shape[0]
    V, D = table.shape
    return _build(B, V, D)(x, table)


# speedup vs baseline: 1.1505x; 1.1153x over previous
import functools

import jax
import jax.numpy as jnp
from jax import lax
from jax.experimental import pallas as pl
from jax.experimental.pallas import tpu as pltpu
from jax.experimental.pallas import tpu_sc as plsc


@functools.cache
def _build(B, V, D):
    info = plsc.get_sparse_core_info()
    NC, NS = info.num_cores, info.num_subcores
    NW = NC * NS
    b_per_w = B // NW
    stage = max(64, -(-V // NS))
    mesh = plsc.VectorSubcoreMesh(core_axis_name="c", subcore_axis_name="s")

    @functools.partial(
        pl.kernel,
        mesh=mesh,
        out_type=jax.ShapeDtypeStruct((B, D), jnp.float32),
        scratch_types=[
            pltpu.VMEM((1, b_per_w), jnp.int32),
            pltpu.VMEM((b_per_w, D), jnp.float32),
            pltpu.VMEM_SHARED((V, D), jnp.float32),
            pltpu.SemaphoreType.DMA,
            pltpu.SemaphoreType.DMA,
        ],
    )
    def k(idx_hbm, table_hbm, out_hbm, idx_v, rows_v, table_sh, isem, gsem):
        cid = lax.axis_index("c")
        sid = lax.axis_index("s")
        wid = sid * NC + cid
        base = wid * b_per_w
        row0 = jnp.minimum(sid * stage, V - stage)
        icopy = pltpu.async_copy(idx_hbm.at[wid], idx_v, isem)
        pltpu.sync_copy(
            table_hbm.at[pl.ds(row0, stage)], table_sh.at[pl.ds(row0, stage)]
        )
        plsc.subcore_barrier()
        icopy.wait()
        pltpu.async_copy(table_sh.at[idx_v.at[0]], rows_v, gsem).wait()
        pltpu.sync_copy(rows_v, out_hbm.at[pl.ds(base, b_per_w)])

    def run(x, table):
        idx = x.astype(jnp.int32).reshape(NW, 1, b_per_w)
        out = k(idx, table)
        return out.reshape(B, 1, D)

    return run


def kernel(x, table):
    B = x.shape[0]
    V, D = table.shape
    return _build(B, V, D)(x, table)


# speedup vs baseline: 1.1824x; 1.0277x over previous
import functools

import jax
import jax.numpy as jnp
from jax import lax
from jax.experimental import pallas as pl
from jax.experimental.pallas import tpu as pltpu
from jax.experimental.pallas import tpu_sc as plsc


@functools.cache
def _build(B, V, D):
    info = plsc.get_sparse_core_info()
    NC, NS = info.num_cores, info.num_subcores
    NW = NC * NS
    b_per_w = B // NW
    stage = max(64, -(-V // NS))
    mesh = plsc.VectorSubcoreMesh(core_axis_name="c", subcore_axis_name="s")

    chunk = 128
    n_chunks = b_per_w // chunk

    @functools.partial(
        pl.kernel,
        mesh=mesh,
        out_type=jax.ShapeDtypeStruct((B, D), jnp.float32),
        scratch_types=[
            pltpu.VMEM((n_chunks, chunk), jnp.int32),
            pltpu.VMEM((n_chunks, chunk, D), jnp.float32),
            pltpu.VMEM_SHARED((V, D), jnp.float32),
            pltpu.SemaphoreType.DMA,
            pltpu.SemaphoreType.DMA((n_chunks,)),
            pltpu.SemaphoreType.DMA,
        ],
    )
    def k(idx_hbm, table_hbm, out_hbm, idx_v, rows_v, table_sh, isem, gsem, wsem):
        cid = lax.axis_index("c")
        sid = lax.axis_index("s")
        wid = sid * NC + cid
        base = wid * b_per_w
        row0 = jnp.minimum(sid * stage, V - stage)
        icopy = pltpu.async_copy(idx_hbm.at[wid], idx_v, isem)
        pltpu.sync_copy(
            table_hbm.at[pl.ds(row0, stage)], table_sh.at[pl.ds(row0, stage)]
        )
        plsc.subcore_barrier()
        icopy.wait()
        gathers = [
            pltpu.async_copy(table_sh.at[idx_v.at[j]], rows_v.at[j], gsem.at[j])
            for j in range(n_chunks)
        ]
        writes = []
        for j in range(n_chunks):
            gathers[j].wait()
            writes.append(
                pltpu.async_copy(
                    rows_v.at[j], out_hbm.at[pl.ds(base + j * chunk, chunk)], wsem
                )
            )
        for w in writes:
            w.wait()

    def run(x, table):
        idx = x.astype(jnp.int32).reshape(NW, n_chunks, chunk)
        out = k(idx, table)
        return out.reshape(B, 1, D)

    return run


def kernel(x, table):
    B = x.shape[0]
    V, D = table.shape
    return _build(B, V, D)(x, table)


# speedup vs baseline: 1.1964x; 1.0119x over previous
import functools

import jax
import jax.numpy as jnp
from jax import lax
from jax.experimental import pallas as pl
from jax.experimental.pallas import tpu as pltpu
from jax.experimental.pallas import tpu_sc as plsc


@functools.cache
def _build(B, V, D):
    info = plsc.get_sparse_core_info()
    NC, NS = info.num_cores, info.num_subcores
    NW = NC * NS
    b_per_w = B // NW
    stage = max(64, -(-V // NS))
    mesh = plsc.VectorSubcoreMesh(core_axis_name="c", subcore_axis_name="s")

    chunk = 64
    n_chunks = b_per_w // chunk

    @functools.partial(
        pl.kernel,
        mesh=mesh,
        out_type=jax.ShapeDtypeStruct((B, D), jnp.float32),
        scratch_types=[
            pltpu.VMEM((n_chunks, chunk), jnp.int32),
            pltpu.VMEM((n_chunks, chunk, D), jnp.float32),
            pltpu.VMEM_SHARED((V, D), jnp.float32),
            pltpu.SemaphoreType.DMA,
            pltpu.SemaphoreType.DMA((n_chunks,)),
            pltpu.SemaphoreType.DMA,
        ],
    )
    def k(idx_hbm, table_hbm, out_hbm, idx_v, rows_v, table_sh, isem, gsem, wsem):
        cid = lax.axis_index("c")
        sid = lax.axis_index("s")
        wid = sid * NC + cid
        base = wid * b_per_w
        row0 = jnp.minimum(sid * stage, V - stage)
        icopy = pltpu.async_copy(idx_hbm.at[wid], idx_v, isem)
        pltpu.sync_copy(
            table_hbm.at[pl.ds(row0, stage)], table_sh.at[pl.ds(row0, stage)]
        )
        plsc.subcore_barrier()
        icopy.wait()
        gathers = [
            pltpu.async_copy(table_sh.at[idx_v.at[j]], rows_v.at[j], gsem.at[j])
            for j in range(n_chunks)
        ]
        writes = []
        for j in range(n_chunks):
            gathers[j].wait()
            writes.append(
                pltpu.async_copy(
                    rows_v.at[j], out_hbm.at[pl.ds(base + j * chunk, chunk)], wsem
                )
            )
        for w in writes:
            w.wait()

    def run(x, table):
        idx = x.astype(jnp.int32).reshape(NW, n_chunks, chunk)
        out = k(idx, table)
        return out.reshape(B, 1, D)

    return run


def kernel(x, table):
    B = x.shape[0]
    V, D = table.shape
    return _build(B, V, D)(x, table)
